# Initial kernel scaffold; baseline (speedup 1.0000x reference)
import jax
import jax.numpy as jnp
from jax.experimental import pallas as pl


def kernel(edge_attr, edge_index, dst_ports, batch, emb_table, W1, b1, W2, b2, W3, b3, Wc1, bc1, Wc2, bc2):
    raise NotImplementedError("write your pallas kernel here")


# speedup vs baseline: 6.7722x; 6.7722x over previous
import functools

import jax
import jax.numpy as jnp
from jax import lax
from jax.experimental import pallas as pl
from jax.experimental.pallas import tpu as pltpu
from jax.experimental.pallas import tpu_sc as plsc

N = 10000
NP = 10240
E = 320000
AD = 16
PD = 64
HID = 192
NCLS = 10
NG = 16
TP = 65536
W = 128

NC, NS = 2, 16
NW = NC * NS
CH = 80
EPT_H = E // NW
EPT_F = E // NS
RPT = NP // NS

_MESH = plsc.VectorSubcoreMesh(
    core_axis_name="c", subcore_axis_name="s", num_cores=NC, num_subcores=NS)


@functools.partial(
    pl.kernel,
    out_type=(
        jax.ShapeDtypeStruct((E, W), jnp.float32),
        jax.ShapeDtypeStruct((NC, NP, W), jnp.float32),
    ),
    mesh=_MESH,
    scratch_types=(
        pltpu.VMEM((CH,), jnp.int32),
        pltpu.VMEM((CH,), jnp.int32),
        pltpu.VMEM((CH, W), jnp.float32),
        pltpu.VMEM_SHARED((NP, W), jnp.float32),
        pltpu.SemaphoreType.DMA,
    ),
)
def _gather_stats(ports_hbm, dst_hbm, table_hbm, zeros_hbm,
                  demb, pe, pidx, didx, emb_v, acc, sem):
    cid = lax.axis_index("c")
    sid = lax.axis_index("s")
    wid = cid * NS + sid
    rows = pl.ds(sid * RPT, RPT)
    pltpu.sync_copy(zeros_hbm.at[rows], acc.at[rows])
    plsc.subcore_barrier()
    base = wid * EPT_H

    def body(k, carry):
        sl = pl.ds(base + k * CH, CH)
        pltpu.sync_copy(ports_hbm.at[sl], pidx)
        pltpu.sync_copy(dst_hbm.at[sl], didx)
        pltpu.async_copy(table_hbm.at[pidx], emb_v, sem).wait()
        pltpu.sync_copy(emb_v, demb.at[sl])
        pltpu.sync_copy(emb_v, acc.at[didx], add=True)
        return carry

    lax.fori_loop(0, EPT_H // CH, body, 0)
    plsc.subcore_barrier()
    pltpu.sync_copy(acc.at[rows], pe.at[cid, rows])


@functools.partial(
    pl.kernel,
    out_type=jax.ShapeDtypeStruct((NC, NP, W), jnp.float32),
    mesh=_MESH,
    scratch_types=(
        pltpu.VMEM((CH,), jnp.int32),
        pltpu.VMEM((CH, W), jnp.float32),
        pltpu.VMEM_SHARED((NP, W), jnp.float32),
    ),
)
def _scatter_rows(vals_hbm, dst_hbm, zeros_hbm, out, didx, rows_v, acc):
    cid = lax.axis_index("c")
    sid = lax.axis_index("s")
    rows = pl.ds(sid * RPT, RPT)
    pltpu.sync_copy(zeros_hbm.at[rows], acc.at[rows])
    plsc.subcore_barrier()
    ebase = sid * EPT_F
    vbase = cid * E + ebase

    def body(k, carry):
        pltpu.sync_copy(dst_hbm.at[pl.ds(ebase + k * CH, CH)], didx)
        pltpu.sync_copy(vals_hbm.at[pl.ds(vbase + k * CH, CH)], rows_v)
        pltpu.sync_copy(rows_v, acc.at[didx], add=True)
        return carry

    lax.fori_loop(0, EPT_F // CH, body, 0)
    plsc.subcore_barrier()
    pltpu.sync_copy(acc.at[rows], out.at[cid, rows])


@functools.partial(
    pl.kernel,
    out_type=jax.ShapeDtypeStruct((NC, NP, W), jnp.float32),
    mesh=_MESH,
    scratch_types=(
        pltpu.VMEM((CH,), jnp.int32),
        pltpu.VMEM((CH,), jnp.int32),
        pltpu.VMEM((CH,), jnp.int32),
        pltpu.VMEM((CH, W), jnp.float32),
        pltpu.VMEM_SHARED((NP, W), jnp.float32),
        pltpu.SemaphoreType.DMA,
    ),
)
def _propagate(x_hbm, src_hbm, dst_hbm, zeros_hbm, out,
               sidx0, sidx, didx, rows_v, acc, sem):
    cid = lax.axis_index("c")
    sid = lax.axis_index("s")
    rows = pl.ds(sid * RPT, RPT)
    pltpu.sync_copy(zeros_hbm.at[rows], acc.at[rows])
    plsc.subcore_barrier()
    ebase = sid * EPT_F
    off = cid * NP

    def body(k, carry):
        sl = pl.ds(ebase + k * CH, CH)
        pltpu.sync_copy(src_hbm.at[sl], sidx0)
        pltpu.sync_copy(dst_hbm.at[sl], didx)
        for j in range(CH // 16):
            s16 = pl.ds(j * 16, 16)
            sidx[s16] = sidx0[s16] + off
        pltpu.async_copy(x_hbm.at[sidx], rows_v, sem).wait()
        pltpu.sync_copy(rows_v, acc.at[didx], add=True)
        return carry

    lax.fori_loop(0, EPT_F // CH, body, 0)
    plsc.subcore_barrier()
    pltpu.sync_copy(acc.at[rows], out.at[cid, rows])


def _mlp_math(xa, xe, w1a, w1e, b1, w2, b2, w3, b3):
    h = jnp.dot(xa, w1a, preferred_element_type=jnp.float32)
    h = h + jnp.dot(xe, w1e, preferred_element_type=jnp.float32)
    h = jnp.maximum(h + b1, 0.0)
    h = jnp.maximum(jnp.dot(h, w2, preferred_element_type=jnp.float32) + b2, 0.0)
    return jnp.dot(h, w3, preferred_element_type=jnp.float32) + b3


def _edge_mlp_body(xa, xe, w1a, w1e, b1, w2, b2, w3, b3, out):
    a = xa[...]
    em = _mlp_math(a, xe[...], w1a[...], w1e[...], b1[...],
                   w2[...], b2[...], w3[...], b3[...])
    lo = em[:, :W]
    hi = jnp.concatenate(
        [em[:, W:], a, jnp.zeros((a.shape[0], W - (HID - W) - AD), jnp.float32)],
        axis=1)
    out[...] = jnp.stack([lo, hi])


def _loop_mlp_body(pe, pc, w1a, w1e, b1, w2, b2, w3, b3, out):
    s = pe[0] + pe[1]
    deg = s[:, PD:PD + 1]
    r = 1.0 / jnp.maximum(deg, 1.0)
    xe = s[:, :PD] * r
    xa = pc[1][:, PD:PD + AD] * r
    em = _mlp_math(xa, xe, w1a[...], w1e[...], b1[...],
                   w2[...], b2[...], w3[...], b3[...])
    lo = em[:, :W]
    hi = jnp.concatenate(
        [em[:, W:], jnp.zeros((em.shape[0], 2 * W - HID), jnp.float32)], axis=1)
    out[...] = jnp.stack([lo, hi])


def _add2_body(a, b, out):
    out[...] = a[...] + b[...]


def _axpy_body(a, b, out):
    out[...] = a[...] + 2.0 * b[...]


def _pool_cls_body(pp, x2, x1, bt, wc1, bc1, wc2, bc2, out, acc):
    i = pl.program_id(0)

    @pl.when(i == 0)
    def _():
        acc[...] = jnp.full((NG, 2 * W), -jnp.inf, jnp.float32)

    x3 = pp[...] + x2[...] + x1[...]
    x = jnp.concatenate([x3[0], x3[1]], axis=1)
    b = bt[...]
    cur = acc[...]
    for g in range(NG):
        xg = jnp.where(b == g, x, -jnp.inf)
        mg = jnp.max(xg, axis=0, keepdims=True)
        seg = (lax.broadcasted_iota(jnp.int32, (NG, 2 * W), 0) == g)
        cur = jnp.where(seg, jnp.maximum(cur, mg), cur)
    acc[...] = cur

    @pl.when(i == pl.num_programs(0) - 1)
    def _():
        gr = acc[...]
        h = jnp.maximum(
            jnp.dot(gr, wc1[...], preferred_element_type=jnp.float32) + bc1[...],
            0.0)
        out[...] = jnp.dot(h, wc2[...], preferred_element_type=jnp.float32) + bc2[...]


_BR_E = 1280
_BR_N = 2048


def _full(shape):
    return pl.BlockSpec(shape, lambda i: (0,) * len(shape))


def _rows(b, w):
    return pl.BlockSpec((b, w), lambda i: (i, 0))


def _rows3(b, w):
    return pl.BlockSpec((NC, b, w), lambda i: (0, i, 0))


_edge_mlp = pl.pallas_call(
    _edge_mlp_body,
    grid=(E // _BR_E,),
    in_specs=[_rows(_BR_E, AD), _rows(_BR_E, W),
              _full((AD, HID)), _full((W, HID)), _full((1, HID)),
              _full((HID, HID)), _full((1, HID)),
              _full((HID, HID)), _full((1, HID))],
    out_specs=_rows3(_BR_E, W),
    out_shape=jax.ShapeDtypeStruct((NC, E, W), jnp.float32),
)

_loop_mlp = pl.pallas_call(
    _loop_mlp_body,
    grid=(NP // _BR_N,),
    in_specs=[_rows3(_BR_N, W), _rows3(_BR_N, W),
              _full((AD, HID)), _full((PD, HID)), _full((1, HID)),
              _full((HID, HID)), _full((1, HID)),
              _full((HID, HID)), _full((1, HID))],
    out_specs=_rows3(_BR_N, W),
    out_shape=jax.ShapeDtypeStruct((NC, NP, W), jnp.float32),
)

_add2 = pl.pallas_call(
    _add2_body,
    grid=(NP // _BR_N,),
    in_specs=[_rows3(_BR_N, W)] * 2,
    out_specs=_rows3(_BR_N, W),
    out_shape=jax.ShapeDtypeStruct((NC, NP, W), jnp.float32),
)

_axpy = pl.pallas_call(
    _axpy_body,
    grid=(NP // _BR_N,),
    in_specs=[_rows3(_BR_N, W)] * 2,
    out_specs=_rows3(_BR_N, W),
    out_shape=jax.ShapeDtypeStruct((NC, NP, W), jnp.float32),
)

_pool_cls = pl.pallas_call(
    _pool_cls_body,
    grid=(NP // _BR_N,),
    in_specs=[_rows3(_BR_N, W)] * 3 + [
        _rows(_BR_N, 1),
        _full((2 * W, HID)), _full((1, HID)),
        _full((HID, NCLS)), _full((1, NCLS))],
    out_specs=_full((NG, NCLS)),
    out_shape=jax.ShapeDtypeStruct((NG, NCLS), jnp.float32),
    scratch_shapes=[pltpu.VMEM((NG, 2 * W), jnp.float32)],
)


def kernel(edge_attr, edge_index, dst_ports, batch, emb_table,
           W1, b1, W2, b2, W3, b3, Wc1, bc1, Wc2, bc2):
    src = edge_index[0]
    dst = edge_index[1]
    zw = jnp.zeros((NP, W), jnp.float32)
    table_aug = jnp.concatenate(
        [emb_table, jnp.ones((TP, 1), jnp.float32),
         jnp.zeros((TP, W - PD - 1), jnp.float32)], axis=1)

    demb, pe = _gather_stats(dst_ports, dst, table_aug, zw)

    w1a = W1[:AD]
    w1e = W1[AD:]
    w1e_pad = jnp.concatenate([w1e, jnp.zeros((W - PD, HID), jnp.float32)])
    b1r, b2r, b3r = b1[None], b2[None], b3[None]

    em2 = _edge_mlp(edge_attr, demb, w1a, w1e_pad, b1r, W2, b2r, W3, b3r)
    pc = _scatter_rows(em2.reshape(NC * E, W), dst, zw)
    em_n2 = _loop_mlp(pe, pc, w1a, w1e, b1r, W2, b2r, W3, b3r)

    x1 = _add2(pc, em_n2)
    pp = _propagate(x1.reshape(NC * NP, W), src, dst, zw)
    x2 = _axpy(pp, x1)
    pp2 = _propagate(x2.reshape(NC * NP, W), src, dst, zw)

    batch_p = jnp.concatenate([batch, jnp.full((NP - N,), NG, jnp.int32)])
    wc1_pad = jnp.concatenate([Wc1, jnp.zeros((2 * W - HID, HID), jnp.float32)])
    out = _pool_cls(pp2, x2, x1, batch_p[:, None],
                    wc1_pad, bc1[None], Wc2, bc2[None])
    return out


# speedup vs baseline: 14.9593x; 2.2089x over previous
import functools

import jax
import jax.numpy as jnp
from jax import lax
from jax.experimental import pallas as pl
from jax.experimental.pallas import tpu as pltpu
from jax.experimental.pallas import tpu_sc as plsc

N = 10000
NP = 10240
E = 320000
AD = 16
PD = 64
HID = 192
NCLS = 10
NG = 16
TP = 65536
W = 128

NC, NS = 2, 16
NW = NC * NS
CH = 80
EPT_H = E // NW
EPT_F = E // NS
RPT = NP // NS

_MESH = plsc.VectorSubcoreMesh(
    core_axis_name="c", subcore_axis_name="s", num_cores=NC, num_subcores=NS)


NBUF = 4
NCH_H = EPT_H // CH
NCH_F = EPT_F // CH


def _cdiv(a, b):
    return (a + b - 1) // b


def _make_stats(ne):
    ept = ne // NW
    nch = ept // CH

    @functools.partial(
        pl.kernel,
        out_type=(
            jax.ShapeDtypeStruct((ne, W), jnp.float32),
            jax.ShapeDtypeStruct((NC, NP, W), jnp.float32),
        ),
        mesh=_MESH,
        scratch_types=(
            pltpu.VMEM((2 * NBUF, CH), jnp.int32),
            pltpu.VMEM((NBUF, CH), jnp.int32),
            pltpu.VMEM((NBUF, CH, W), jnp.float32),
            pltpu.VMEM_SHARED((NP, W), jnp.float32),
            [pltpu.SemaphoreType.DMA] * NBUF,
            [pltpu.SemaphoreType.DMA] * NBUF,
            [pltpu.SemaphoreType.DMA] * (2 * NBUF),
            pltpu.SemaphoreType.DMA,
        ),
    )
    def _gather_stats(ports_hbm, dst_hbm, table_hbm, init_hbm,
                      demb, pe, pring, dring, ring, acc, gsem, dsem, isem,
                      wsem):
        cid = lax.axis_index("c")
        sid = lax.axis_index("s")
        wid = cid * NS + sid
        rows = pl.ds(sid * RPT, RPT)
        pltpu.sync_copy(init_hbm.at[pl.ds(cid * NP + sid * RPT, RPT)],
                        acc.at[rows])
        plsc.subcore_barrier()
        base = wid * ept

        def echunk(k):
            return pl.ds(base + k * CH, CH)

        for j in range(NBUF):
            pltpu.sync_copy(ports_hbm.at[echunk(j)], pring.at[j])
            pltpu.sync_copy(dst_hbm.at[echunk(j)], dring.at[j])
            pltpu.async_copy(table_hbm.at[pring.at[j]], ring.at[j], gsem[j])
            pltpu.async_copy(ports_hbm.at[echunk(j + NBUF)],
                             pring.at[j + NBUF], isem[j + NBUF])

        def body(ko, carry):
            for jj in range(2 * NBUF):
                j = jj % NBUF
                jp = (jj + NBUF) % (2 * NBUF)
                k = ko * 2 * NBUF + jj

                @pl.when(k < nch)
                def _():
                    pltpu.make_async_copy(table_hbm.at[pl.ds(0, CH)],
                                          ring.at[j], gsem[j]).wait()
                    wr = pltpu.async_copy(ring.at[j], demb.at[echunk(k)], wsem)

                    @pl.when(k >= NBUF)
                    def _():
                        pltpu.make_async_copy(dst_hbm.at[pl.ds(0, CH)],
                                              dring.at[j], dsem[j]).wait()

                    pltpu.sync_copy(ring.at[j], acc.at[dring.at[j]], add=True)

                    @pl.when(k + NBUF < nch)
                    def _():
                        pltpu.async_copy(dst_hbm.at[echunk(k + NBUF)],
                                         dring.at[j], dsem[j])

                    wr.wait()

                    @pl.when(k + NBUF < nch)
                    def _():
                        pltpu.make_async_copy(ports_hbm.at[pl.ds(0, CH)],
                                              pring.at[jp], isem[jp]).wait()
                        pltpu.async_copy(table_hbm.at[pring.at[jp]],
                                         ring.at[j], gsem[j])

                    @pl.when(k + 2 * NBUF < nch)
                    def _():
                        pltpu.async_copy(ports_hbm.at[echunk(k + 2 * NBUF)],
                                         pring.at[jj], isem[jj])
            return carry

        lax.fori_loop(0, _cdiv(nch, 2 * NBUF), body, 0)
        plsc.subcore_barrier()
        pltpu.sync_copy(acc.at[rows], pe.at[cid, rows])

    return _gather_stats


EA = 192000
EB = E - EA
_stats_a = _make_stats(EA)
_stats_b = _make_stats(EB)


def _make_scatter(ne):
    ept = ne // NS
    nch = ept // CH

    @functools.partial(
        pl.kernel,
        out_type=jax.ShapeDtypeStruct((NC, NP, W), jnp.float32),
        mesh=_MESH,
        scratch_types=(
            pltpu.VMEM((NBUF, CH), jnp.int32),
            pltpu.VMEM((NBUF, CH, W), jnp.float32),
            pltpu.VMEM_SHARED((NP, W), jnp.float32),
            [pltpu.SemaphoreType.DMA] * NBUF,
            [pltpu.SemaphoreType.DMA] * NBUF,
        ),
    )
    def _scatter_rows(vals_hbm, dst_hbm, init_hbm, out, dring, ring, acc,
                      gsem, dsem):
        cid = lax.axis_index("c")
        sid = lax.axis_index("s")
        rows = pl.ds(sid * RPT, RPT)
        pltpu.sync_copy(init_hbm.at[pl.ds(cid * NP + sid * RPT, RPT)],
                        acc.at[rows])
        plsc.subcore_barrier()
        ebase = sid * ept
        vbase = cid * ne + ebase
        for j in range(NBUF):
            pltpu.sync_copy(dst_hbm.at[pl.ds(ebase + j * CH, CH)], dring.at[j])
            pltpu.async_copy(vals_hbm.at[pl.ds(vbase + j * CH, CH)],
                             ring.at[j], gsem[j])

        def body(ko, carry):
            for j in range(NBUF):
                k = ko * NBUF + j

                @pl.when(k < nch)
                def _():
                    pltpu.make_async_copy(vals_hbm.at[pl.ds(0, CH)],
                                          ring.at[j], gsem[j]).wait()

                    @pl.when(k >= NBUF)
                    def _():
                        pltpu.make_async_copy(dst_hbm.at[pl.ds(0, CH)],
                                              dring.at[j], dsem[j]).wait()

                    pltpu.sync_copy(ring.at[j], acc.at[dring.at[j]], add=True)

                    @pl.when(k + NBUF < nch)
                    def _():
                        pltpu.async_copy(
                            dst_hbm.at[pl.ds(ebase + (k + NBUF) * CH, CH)],
                            dring.at[j], dsem[j])
                        pltpu.async_copy(
                            vals_hbm.at[pl.ds(vbase + (k + NBUF) * CH, CH)],
                            ring.at[j], gsem[j])
            return carry

        lax.fori_loop(0, _cdiv(nch, NBUF), body, 0)
        plsc.subcore_barrier()
        pltpu.sync_copy(acc.at[rows], out.at[cid, rows])

    return _scatter_rows


_scatter_a = _make_scatter(EA)
_scatter_b = _make_scatter(EB)


@functools.partial(
    pl.kernel,
    out_type=jax.ShapeDtypeStruct((NC, NP, W), jnp.float32),
    mesh=_MESH,
    scratch_types=(
        pltpu.VMEM((2 * NBUF, CH), jnp.int32),
        pltpu.VMEM((NBUF, CH), jnp.int32),
        pltpu.VMEM((NBUF, CH, W), jnp.float32),
        pltpu.VMEM_SHARED((NP, W), jnp.float32),
        [pltpu.SemaphoreType.DMA] * NBUF,
        [pltpu.SemaphoreType.DMA] * NBUF,
        [pltpu.SemaphoreType.DMA] * (2 * NBUF),
    ),
)
def _propagate(x_hbm, src2_hbm, dst_hbm, out,
               sring, dring, ring, acc, gsem, dsem, isem):
    cid = lax.axis_index("c")
    sid = lax.axis_index("s")
    rows = pl.ds(sid * RPT, RPT)
    pltpu.sync_copy(x_hbm.at[pl.ds(cid * NP + sid * RPT, RPT)], acc.at[rows])
    plsc.subcore_barrier()
    ebase = sid * EPT_F
    sbase = cid * E + ebase

    for j in range(NBUF):
        pltpu.sync_copy(src2_hbm.at[pl.ds(sbase + j * CH, CH)], sring.at[j])
        pltpu.sync_copy(dst_hbm.at[pl.ds(ebase + j * CH, CH)], dring.at[j])
        pltpu.async_copy(x_hbm.at[sring.at[j]], ring.at[j], gsem[j])
        pltpu.async_copy(src2_hbm.at[pl.ds(sbase + (j + NBUF) * CH, CH)],
                         sring.at[j + NBUF], isem[j + NBUF])

    def body(ko, carry):
        for jj in range(2 * NBUF):
            j = jj % NBUF
            jp = (jj + NBUF) % (2 * NBUF)
            k = ko * 2 * NBUF + jj

            @pl.when(k < NCH_F)
            def _():
                pltpu.make_async_copy(x_hbm.at[pl.ds(0, CH)],
                                      ring.at[j], gsem[j]).wait()

                @pl.when(k >= NBUF)
                def _():
                    pltpu.make_async_copy(dst_hbm.at[pl.ds(0, CH)],
                                          dring.at[j], dsem[j]).wait()

                pltpu.sync_copy(ring.at[j], acc.at[dring.at[j]], add=True)

                @pl.when(k + NBUF < NCH_F)
                def _():
                    pltpu.async_copy(
                        dst_hbm.at[pl.ds(ebase + (k + NBUF) * CH, CH)],
                        dring.at[j], dsem[j])
                    pltpu.make_async_copy(src2_hbm.at[pl.ds(0, CH)],
                                          sring.at[jp], isem[jp]).wait()
                    pltpu.async_copy(x_hbm.at[sring.at[jp]], ring.at[j], gsem[j])

                @pl.when(k + 2 * NBUF < NCH_F)
                def _():
                    pltpu.async_copy(
                        src2_hbm.at[pl.ds(sbase + (k + 2 * NBUF) * CH, CH)],
                        sring.at[jj], isem[jj])
        return carry

    lax.fori_loop(0, _cdiv(NCH_F, 2 * NBUF), body, 0)
    plsc.subcore_barrier()
    pltpu.sync_copy(acc.at[rows], out.at[cid, rows])


def _mlp_math(xa, xe, w1a, w1e, b1, w2, b2, w3, b3):
    h = jnp.dot(xa, w1a, preferred_element_type=jnp.float32)
    h = h + jnp.dot(xe, w1e, preferred_element_type=jnp.float32)
    h = jnp.maximum(h + b1, 0.0)
    h = jnp.maximum(jnp.dot(h, w2, preferred_element_type=jnp.float32) + b2, 0.0)
    return jnp.dot(h, w3, preferred_element_type=jnp.float32) + b3


def _edge_mlp_body(xa, xe, w1a, w1e, b1, w2, b2, w3, b3, out):
    a = xa[...]
    em = _mlp_math(a, xe[...], w1a[...], w1e[...], b1[...],
                   w2[...], b2[...], w3[...], b3[...])
    lo = em[:, :W]
    hi = jnp.concatenate(
        [em[:, W:], a, jnp.zeros((a.shape[0], W - (HID - W) - AD), jnp.float32)],
        axis=1)
    out[...] = jnp.stack([lo, hi])


def _loop_mlp_body(pe, pc, w1a, w1e, b1, w2, b2, w3, b3, out):
    s = pe[0] + pe[1]
    deg = s[:, PD:PD + 1]
    r = 1.0 / jnp.maximum(deg, 1.0)
    xe = s[:, :PD] * r
    xa = pc[1][:, PD:PD + AD] * r
    em = _mlp_math(xa, xe, w1a[...], w1e[...], b1[...],
                   w2[...], b2[...], w3[...], b3[...])
    lo = em[:, :W]
    hi = jnp.concatenate(
        [em[:, W:], jnp.zeros((em.shape[0], 2 * W - HID), jnp.float32)], axis=1)
    out[...] = jnp.stack([lo, hi]) + pc[...]


def _add2_body(a, b, out):
    out[...] = a[...] + b[...]


def _axpy_body(a, b, out):
    out[...] = a[...] + 2.0 * b[...]


def _pool_cls_body(y3, x1, bt, wc1, bc1, wc2, bc2, out, acc):
    i = pl.program_id(0)

    @pl.when(i == 0)
    def _():
        acc[...] = jnp.full((NG, 2 * W), -jnp.inf, jnp.float32)

    x3 = y3[...] + x1[...]
    x = jnp.concatenate([x3[0], x3[1]], axis=1)
    b = bt[...]
    cur = acc[...]
    for g in range(NG):
        xg = jnp.where(b == g, x, -jnp.inf)
        mg = jnp.max(xg, axis=0, keepdims=True)
        seg = (lax.broadcasted_iota(jnp.int32, (NG, 2 * W), 0) == g)
        cur = jnp.where(seg, jnp.maximum(cur, mg), cur)
    acc[...] = cur

    @pl.when(i == pl.num_programs(0) - 1)
    def _():
        gr = acc[...]
        h = jnp.maximum(
            jnp.dot(gr, wc1[...], preferred_element_type=jnp.float32) + bc1[...],
            0.0)
        out[...] = jnp.dot(h, wc2[...], preferred_element_type=jnp.float32) + bc2[...]


_BR_E = 1280
_BR_N = 2048


def _full(shape):
    return pl.BlockSpec(shape, lambda i: (0,) * len(shape))


def _rows(b, w):
    return pl.BlockSpec((b, w), lambda i: (i, 0))


def _rows3(b, w):
    return pl.BlockSpec((NC, b, w), lambda i: (0, i, 0))


def _make_edge_mlp(ne):
    return pl.pallas_call(
        _edge_mlp_body,
        grid=(ne // _BR_E,),
        in_specs=[_rows(_BR_E, AD), _rows(_BR_E, W),
                  _full((AD, HID)), _full((W, HID)), _full((1, HID)),
                  _full((HID, HID)), _full((1, HID)),
                  _full((HID, HID)), _full((1, HID))],
        out_specs=_rows3(_BR_E, W),
        out_shape=jax.ShapeDtypeStruct((NC, ne, W), jnp.float32),
    )


_edge_mlp_a = _make_edge_mlp(EA)
_edge_mlp_b = _make_edge_mlp(EB)

_loop_mlp = pl.pallas_call(
    _loop_mlp_body,
    grid=(NP // _BR_N,),
    in_specs=[_rows3(_BR_N, W), _rows3(_BR_N, W),
              _full((AD, HID)), _full((PD, HID)), _full((1, HID)),
              _full((HID, HID)), _full((1, HID)),
              _full((HID, HID)), _full((1, HID))],
    out_specs=_rows3(_BR_N, W),
    out_shape=jax.ShapeDtypeStruct((NC, NP, W), jnp.float32),
)

_add2 = pl.pallas_call(
    _add2_body,
    grid=(NP // _BR_N,),
    in_specs=[_rows3(_BR_N, W)] * 2,
    out_specs=_rows3(_BR_N, W),
    out_shape=jax.ShapeDtypeStruct((NC, NP, W), jnp.float32),
)

_axpy = pl.pallas_call(
    _axpy_body,
    grid=(NP // _BR_N,),
    in_specs=[_rows3(_BR_N, W)] * 2,
    out_specs=_rows3(_BR_N, W),
    out_shape=jax.ShapeDtypeStruct((NC, NP, W), jnp.float32),
)

_pool_cls = pl.pallas_call(
    _pool_cls_body,
    grid=(NP // _BR_N,),
    in_specs=[_rows3(_BR_N, W)] * 2 + [
        _rows(_BR_N, 1),
        _full((2 * W, HID)), _full((1, HID)),
        _full((HID, NCLS)), _full((1, NCLS))],
    out_specs=_full((NG, NCLS)),
    out_shape=jax.ShapeDtypeStruct((NG, NCLS), jnp.float32),
    scratch_shapes=[pltpu.VMEM((NG, 2 * W), jnp.float32)],
)


def kernel(edge_attr, edge_index, dst_ports, batch, emb_table,
           W1, b1, W2, b2, W3, b3, Wc1, bc1, Wc2, bc2):
    src = edge_index[0]
    dst = edge_index[1]
    zw2 = jnp.zeros((NC * NP, W), jnp.float32)
    table_aug = jnp.concatenate(
        [emb_table, jnp.ones((TP, 1), jnp.float32),
         jnp.zeros((TP, W - PD - 1), jnp.float32)], axis=1)

    demb_a, pe_a = _stats_a(dst_ports[:EA], dst[:EA], table_aug, zw2)
    demb_b, pe = _stats_b(dst_ports[EA:], dst[EA:], table_aug,
                          pe_a.reshape(NC * NP, W))

    w1a = W1[:AD]
    w1e = W1[AD:]
    w1e_pad = jnp.concatenate([w1e, jnp.zeros((W - PD, HID), jnp.float32)])
    b1r, b2r, b3r = b1[None], b2[None], b3[None]

    em2_a = _edge_mlp_a(edge_attr[:EA], demb_a, w1a, w1e_pad, b1r,
                        W2, b2r, W3, b3r)
    em2_b = _edge_mlp_b(edge_attr[EA:], demb_b, w1a, w1e_pad, b1r,
                        W2, b2r, W3, b3r)
    p1 = _scatter_a(em2_a.reshape(NC * EA, W), dst[:EA], zw2)
    pc = _scatter_b(em2_b.reshape(NC * EB, W), dst[EA:],
                    p1.reshape(NC * NP, W))
    em_n2 = _loop_mlp(pe, pc, w1a, w1e, b1r, W2, b2r, W3, b3r)

    x1 = em_n2
    src2 = jnp.concatenate([src, src + NP])
    y2 = _propagate(x1.reshape(NC * NP, W), src2, dst)
    x2 = _add2(y2, x1)
    y3 = _propagate(x2.reshape(NC * NP, W), src2, dst)

    batch_p = jnp.concatenate([batch, jnp.full((NP - N,), NG, jnp.int32)])
    wc1_pad = jnp.concatenate([Wc1, jnp.zeros((2 * W - HID, HID), jnp.float32)])
    out = _pool_cls(y3, x1, batch_p[:, None],
                    wc1_pad, bc1[None], Wc2, bc2[None])
    return out


# speedup vs baseline: 15.9712x; 1.0676x over previous
import functools

import jax
import jax.numpy as jnp
from jax import lax
from jax.experimental import pallas as pl
from jax.experimental.pallas import tpu as pltpu
from jax.experimental.pallas import tpu_sc as plsc

N = 10000
NP = 10240
E = 320000
AD = 16
PD = 64
HID = 192
NCLS = 10
NG = 16
TP = 65536
W = 128

NC, NS = 2, 16
NW = NC * NS
CH = 80
EPT_H = E // NW
EPT_F = E // NS
RPT = NP // NS

_MESH = plsc.VectorSubcoreMesh(
    core_axis_name="c", subcore_axis_name="s", num_cores=NC, num_subcores=NS)


NBUF = 4
NCH_H = EPT_H // CH
NCH_F = EPT_F // CH


def _cdiv(a, b):
    return (a + b - 1) // b


def _make_stats(ne):
    ept = ne // NW
    nch = ept // CH

    @functools.partial(
        pl.kernel,
        out_type=(
            jax.ShapeDtypeStruct((ne, W), jnp.float32),
            jax.ShapeDtypeStruct((NC, NP, W), jnp.float32),
        ),
        mesh=_MESH,
        scratch_types=(
            pltpu.VMEM((2 * NBUF, CH), jnp.int32),
            pltpu.VMEM((NBUF, CH), jnp.int32),
            pltpu.VMEM((NBUF, CH, W), jnp.float32),
            pltpu.VMEM_SHARED((NP, W), jnp.float32),
            [pltpu.SemaphoreType.DMA] * NBUF,
            [pltpu.SemaphoreType.DMA] * NBUF,
            [pltpu.SemaphoreType.DMA] * (2 * NBUF),
            pltpu.SemaphoreType.DMA,
        ),
    )
    def _gather_stats(ports_hbm, dst_hbm, table_hbm, init_hbm,
                      demb, pe, pring, dring, ring, acc, gsem, dsem, isem,
                      wsem):
        cid = lax.axis_index("c")
        sid = lax.axis_index("s")
        wid = cid * NS + sid
        rows = pl.ds(sid * RPT, RPT)
        pltpu.sync_copy(init_hbm.at[pl.ds(cid * NP + sid * RPT, RPT)],
                        acc.at[rows])
        plsc.subcore_barrier()
        base = wid * ept

        def echunk(k):
            return pl.ds(base + k * CH, CH)

        for j in range(NBUF):
            pltpu.sync_copy(ports_hbm.at[echunk(j)], pring.at[j])
            pltpu.sync_copy(dst_hbm.at[echunk(j)], dring.at[j])
            pltpu.async_copy(table_hbm.at[pring.at[j]], ring.at[j], gsem[j])
            pltpu.async_copy(ports_hbm.at[echunk(j + NBUF)],
                             pring.at[j + NBUF], isem[j + NBUF])

        def body(ko, carry):
            for jj in range(2 * NBUF):
                j = jj % NBUF
                jp = (jj + NBUF) % (2 * NBUF)
                k = ko * 2 * NBUF + jj

                @pl.when(k < nch)
                def _():
                    pltpu.make_async_copy(table_hbm.at[pl.ds(0, CH)],
                                          ring.at[j], gsem[j]).wait()
                    wr = pltpu.async_copy(ring.at[j], demb.at[echunk(k)], wsem)

                    @pl.when(k >= NBUF)
                    def _():
                        pltpu.make_async_copy(dst_hbm.at[pl.ds(0, CH)],
                                              dring.at[j], dsem[j]).wait()

                    pltpu.sync_copy(ring.at[j], acc.at[dring.at[j]], add=True)

                    @pl.when(k + NBUF < nch)
                    def _():
                        pltpu.async_copy(dst_hbm.at[echunk(k + NBUF)],
                                         dring.at[j], dsem[j])

                    wr.wait()

                    @pl.when(k + NBUF < nch)
                    def _():
                        pltpu.make_async_copy(ports_hbm.at[pl.ds(0, CH)],
                                              pring.at[jp], isem[jp]).wait()
                        pltpu.async_copy(table_hbm.at[pring.at[jp]],
                                         ring.at[j], gsem[j])

                    @pl.when(k + 2 * NBUF < nch)
                    def _():
                        pltpu.async_copy(ports_hbm.at[echunk(k + 2 * NBUF)],
                                         pring.at[jj], isem[jj])
            return carry

        lax.fori_loop(0, _cdiv(nch, 2 * NBUF), body, 0)
        plsc.subcore_barrier()
        pltpu.sync_copy(acc.at[rows], pe.at[cid, rows])

    return _gather_stats


EA = 192000
EB = E - EA
_stats_a = _make_stats(EA)
_stats_b = _make_stats(EB)


def _make_scatter(ne):
    ept = ne // NS
    nch = ept // CH

    @functools.partial(
        pl.kernel,
        out_type=jax.ShapeDtypeStruct((NC, NP, W), jnp.float32),
        mesh=_MESH,
        scratch_types=(
            pltpu.VMEM((NBUF, CH), jnp.int32),
            pltpu.VMEM((NBUF, CH, W), jnp.float32),
            pltpu.VMEM_SHARED((NP, W), jnp.float32),
            [pltpu.SemaphoreType.DMA] * NBUF,
            [pltpu.SemaphoreType.DMA] * NBUF,
        ),
    )
    def _scatter_rows(vals_hbm, dst_hbm, init_hbm, out, dring, ring, acc,
                      gsem, dsem):
        cid = lax.axis_index("c")
        sid = lax.axis_index("s")
        rows = pl.ds(sid * RPT, RPT)
        pltpu.sync_copy(init_hbm.at[pl.ds(cid * NP + sid * RPT, RPT)],
                        acc.at[rows])
        plsc.subcore_barrier()
        ebase = sid * ept
        vbase = cid * ne + ebase
        for j in range(NBUF):
            pltpu.sync_copy(dst_hbm.at[pl.ds(ebase + j * CH, CH)], dring.at[j])
            pltpu.async_copy(vals_hbm.at[pl.ds(vbase + j * CH, CH)],
                             ring.at[j], gsem[j])

        def body(ko, carry):
            for j in range(NBUF):
                k = ko * NBUF + j

                @pl.when(k < nch)
                def _():
                    pltpu.make_async_copy(vals_hbm.at[pl.ds(0, CH)],
                                          ring.at[j], gsem[j]).wait()

                    @pl.when(k >= NBUF)
                    def _():
                        pltpu.make_async_copy(dst_hbm.at[pl.ds(0, CH)],
                                              dring.at[j], dsem[j]).wait()

                    pltpu.sync_copy(ring.at[j], acc.at[dring.at[j]], add=True)

                    @pl.when(k + NBUF < nch)
                    def _():
                        pltpu.async_copy(
                            dst_hbm.at[pl.ds(ebase + (k + NBUF) * CH, CH)],
                            dring.at[j], dsem[j])
                        pltpu.async_copy(
                            vals_hbm.at[pl.ds(vbase + (k + NBUF) * CH, CH)],
                            ring.at[j], gsem[j])
            return carry

        lax.fori_loop(0, _cdiv(nch, NBUF), body, 0)
        plsc.subcore_barrier()
        pltpu.sync_copy(acc.at[rows], out.at[cid, rows])

    return _scatter_rows


_scatter_a = _make_scatter(EA)
_scatter_b = _make_scatter(EB)


@functools.partial(
    pl.kernel,
    out_type=jax.ShapeDtypeStruct((NC, NP, W), jnp.float32),
    mesh=_MESH,
    scratch_types=(
        pltpu.VMEM((2 * NBUF, CH), jnp.int32),
        pltpu.VMEM((NBUF, CH), jnp.int32),
        pltpu.VMEM((NBUF, CH, W), jnp.float32),
        pltpu.VMEM_SHARED((NP, W), jnp.float32),
        [pltpu.SemaphoreType.DMA] * NBUF,
        [pltpu.SemaphoreType.DMA] * NBUF,
        [pltpu.SemaphoreType.DMA] * (2 * NBUF),
    ),
)
def _propagate(x_hbm, src2_hbm, dst_hbm, out,
               sring, dring, ring, acc, gsem, dsem, isem):
    cid = lax.axis_index("c")
    sid = lax.axis_index("s")
    rows = pl.ds(sid * RPT, RPT)
    pltpu.sync_copy(x_hbm.at[pl.ds(cid * NP + sid * RPT, RPT)], acc.at[rows])
    plsc.subcore_barrier()
    ebase = sid * EPT_F
    sbase = cid * E + ebase

    for j in range(NBUF):
        pltpu.sync_copy(src2_hbm.at[pl.ds(sbase + j * CH, CH)], sring.at[j])
        pltpu.sync_copy(dst_hbm.at[pl.ds(ebase + j * CH, CH)], dring.at[j])
        pltpu.async_copy(x_hbm.at[sring.at[j]], ring.at[j], gsem[j])
        pltpu.async_copy(src2_hbm.at[pl.ds(sbase + (j + NBUF) * CH, CH)],
                         sring.at[j + NBUF], isem[j + NBUF])

    def body(ko, carry):
        for jj in range(2 * NBUF):
            j = jj % NBUF
            jp = (jj + NBUF) % (2 * NBUF)
            k = ko * 2 * NBUF + jj

            @pl.when(k < NCH_F)
            def _():
                pltpu.make_async_copy(x_hbm.at[pl.ds(0, CH)],
                                      ring.at[j], gsem[j]).wait()

                @pl.when(k >= NBUF)
                def _():
                    pltpu.make_async_copy(dst_hbm.at[pl.ds(0, CH)],
                                          dring.at[j], dsem[j]).wait()

                pltpu.sync_copy(ring.at[j], acc.at[dring.at[j]], add=True)

                @pl.when(k + NBUF < NCH_F)
                def _():
                    pltpu.async_copy(
                        dst_hbm.at[pl.ds(ebase + (k + NBUF) * CH, CH)],
                        dring.at[j], dsem[j])
                    pltpu.make_async_copy(src2_hbm.at[pl.ds(0, CH)],
                                          sring.at[jp], isem[jp]).wait()
                    pltpu.async_copy(x_hbm.at[sring.at[jp]], ring.at[j], gsem[j])

                @pl.when(k + 2 * NBUF < NCH_F)
                def _():
                    pltpu.async_copy(
                        src2_hbm.at[pl.ds(sbase + (k + 2 * NBUF) * CH, CH)],
                        sring.at[jj], isem[jj])
        return carry

    lax.fori_loop(0, _cdiv(NCH_F, 2 * NBUF), body, 0)
    plsc.subcore_barrier()
    pltpu.sync_copy(acc.at[rows], out.at[cid, rows])


def _mlp_math(xa, xe, w1a, w1e, b1, w2, b2, w3, b3):
    h = jnp.dot(xa, w1a, preferred_element_type=jnp.float32)
    h = h + jnp.dot(xe, w1e, preferred_element_type=jnp.float32)
    h = jnp.maximum(h + b1, 0.0)
    h = jnp.maximum(jnp.dot(h, w2, preferred_element_type=jnp.float32) + b2, 0.0)
    return jnp.dot(h, w3, preferred_element_type=jnp.float32) + b3


def _edge_mlp_body(xa, xe, w1a, w1e, b1, w2, b2, w3, b3, out):
    a = xa[...]
    em = _mlp_math(a, xe[...], w1a[...], w1e[...], b1[...],
                   w2[...], b2[...], w3[...], b3[...])
    lo = em[:, :W]
    hi = jnp.concatenate(
        [em[:, W:], a, jnp.zeros((a.shape[0], W - (HID - W) - AD), jnp.float32)],
        axis=1)
    out[...] = jnp.stack([lo, hi])


def _loop_mlp_body(pe, pc, w1a, w1e, b1, w2, b2, w3, b3, out):
    s = pe[0] + pe[1]
    deg = s[:, PD:PD + 1]
    r = 1.0 / jnp.maximum(deg, 1.0)
    xe = s[:, :PD] * r
    xa = pc[1][:, PD:PD + AD] * r
    em = _mlp_math(xa, xe, w1a[...], w1e[...], b1[...],
                   w2[...], b2[...], w3[...], b3[...])
    lo = em[:, :W]
    hi = jnp.concatenate(
        [em[:, W:], jnp.zeros((em.shape[0], 2 * W - HID), jnp.float32)], axis=1)
    out[...] = jnp.stack([lo, hi]) + pc[...]


def _add2_body(a, b, out):
    out[...] = a[...] + b[...]


def _axpy_body(a, b, out):
    out[...] = a[...] + 2.0 * b[...]


def _pool_cls_body(y3, x1, bt, wc1, bc1, wc2, bc2, out, acc):
    i = pl.program_id(0)

    @pl.when(i == 0)
    def _():
        acc[...] = jnp.full((NG, 2 * W), -jnp.inf, jnp.float32)

    x3 = y3[...] + x1[...]
    x = jnp.concatenate([x3[0], x3[1]], axis=1)
    b = bt[...]
    cur = acc[...]
    for g in range(NG):
        xg = jnp.where(b == g, x, -jnp.inf)
        mg = jnp.max(xg, axis=0, keepdims=True)
        seg = (lax.broadcasted_iota(jnp.int32, (NG, 2 * W), 0) == g)
        cur = jnp.where(seg, jnp.maximum(cur, mg), cur)
    acc[...] = cur

    @pl.when(i == pl.num_programs(0) - 1)
    def _():
        gr = acc[...]
        h = jnp.maximum(
            jnp.dot(gr, wc1[...], preferred_element_type=jnp.float32) + bc1[...],
            0.0)
        out[...] = jnp.dot(h, wc2[...], preferred_element_type=jnp.float32) + bc2[...]


_BR_E = 2560
_BR_N = 2048


def _full(shape):
    return pl.BlockSpec(shape, lambda i: (0,) * len(shape))


def _rows(b, w):
    return pl.BlockSpec((b, w), lambda i: (i, 0))


def _rows3(b, w):
    return pl.BlockSpec((NC, b, w), lambda i: (0, i, 0))


def _make_edge_mlp(ne):
    return pl.pallas_call(
        _edge_mlp_body,
        grid=(ne // _BR_E,),
        in_specs=[_rows(_BR_E, AD), _rows(_BR_E, W),
                  _full((AD, HID)), _full((W, HID)), _full((1, HID)),
                  _full((HID, HID)), _full((1, HID)),
                  _full((HID, HID)), _full((1, HID))],
        out_specs=_rows3(_BR_E, W),
        out_shape=jax.ShapeDtypeStruct((NC, ne, W), jnp.float32),
    )


_edge_mlp_a = _make_edge_mlp(EA)
_edge_mlp_b = _make_edge_mlp(EB)

_loop_mlp = pl.pallas_call(
    _loop_mlp_body,
    grid=(NP // _BR_N,),
    in_specs=[_rows3(_BR_N, W), _rows3(_BR_N, W),
              _full((AD, HID)), _full((PD, HID)), _full((1, HID)),
              _full((HID, HID)), _full((1, HID)),
              _full((HID, HID)), _full((1, HID))],
    out_specs=_rows3(_BR_N, W),
    out_shape=jax.ShapeDtypeStruct((NC, NP, W), jnp.float32),
)

_add2 = pl.pallas_call(
    _add2_body,
    grid=(NP // _BR_N,),
    in_specs=[_rows3(_BR_N, W)] * 2,
    out_specs=_rows3(_BR_N, W),
    out_shape=jax.ShapeDtypeStruct((NC, NP, W), jnp.float32),
)

_axpy = pl.pallas_call(
    _axpy_body,
    grid=(NP // _BR_N,),
    in_specs=[_rows3(_BR_N, W)] * 2,
    out_specs=_rows3(_BR_N, W),
    out_shape=jax.ShapeDtypeStruct((NC, NP, W), jnp.float32),
)

_pool_cls = pl.pallas_call(
    _pool_cls_body,
    grid=(NP // _BR_N,),
    in_specs=[_rows3(_BR_N, W)] * 2 + [
        _rows(_BR_N, 1),
        _full((2 * W, HID)), _full((1, HID)),
        _full((HID, NCLS)), _full((1, NCLS))],
    out_specs=_full((NG, NCLS)),
    out_shape=jax.ShapeDtypeStruct((NG, NCLS), jnp.float32),
    scratch_shapes=[pltpu.VMEM((NG, 2 * W), jnp.float32)],
)


def kernel(edge_attr, edge_index, dst_ports, batch, emb_table,
           W1, b1, W2, b2, W3, b3, Wc1, bc1, Wc2, bc2):
    src = edge_index[0]
    dst = edge_index[1]
    zw2 = jnp.zeros((NC * NP, W), jnp.float32)
    table_aug = jnp.concatenate(
        [emb_table, jnp.ones((TP, 1), jnp.float32),
         jnp.zeros((TP, W - PD - 1), jnp.float32)], axis=1)

    demb_a, pe_a = _stats_a(dst_ports[:EA], dst[:EA], table_aug, zw2)
    demb_b, pe = _stats_b(dst_ports[EA:], dst[EA:], table_aug,
                          pe_a.reshape(NC * NP, W))

    w1a = W1[:AD]
    w1e = W1[AD:]
    w1e_pad = jnp.concatenate([w1e, jnp.zeros((W - PD, HID), jnp.float32)])
    b1r, b2r, b3r = b1[None], b2[None], b3[None]

    em2_a = _edge_mlp_a(edge_attr[:EA], demb_a, w1a, w1e_pad, b1r,
                        W2, b2r, W3, b3r)
    em2_b = _edge_mlp_b(edge_attr[EA:], demb_b, w1a, w1e_pad, b1r,
                        W2, b2r, W3, b3r)
    p1 = _scatter_a(em2_a.reshape(NC * EA, W), dst[:EA], zw2)
    pc = _scatter_b(em2_b.reshape(NC * EB, W), dst[EA:],
                    p1.reshape(NC * NP, W))
    em_n2 = _loop_mlp(pe, pc, w1a, w1e, b1r, W2, b2r, W3, b3r)

    x1 = em_n2
    src2 = jnp.concatenate([src, src + NP])
    y2 = _propagate(x1.reshape(NC * NP, W), src2, dst)
    x2 = _add2(y2, x1)
    y3 = _propagate(x2.reshape(NC * NP, W), src2, dst)

    batch_p = jnp.concatenate([batch, jnp.full((NP - N,), NG, jnp.int32)])
    wc1_pad = jnp.concatenate([Wc1, jnp.zeros((2 * W - HID, HID), jnp.float32)])
    out = _pool_cls(y3, x1, batch_p[:, None],
                    wc1_pad, bc1[None], Wc2, bc2[None])
    return out


# speedup vs baseline: 16.5450x; 1.0359x over previous
import functools

import jax
import jax.numpy as jnp
from jax import lax
from jax.experimental import pallas as pl
from jax.experimental.pallas import tpu as pltpu
from jax.experimental.pallas import tpu_sc as plsc

N = 10000
NP = 10240
E = 320000
AD = 16
PD = 64
HID = 192
NCLS = 10
NG = 16
TP = 65536
W = 128

NC, NS = 2, 16
NW = NC * NS
CH = 80
EPT_H = E // NW
EPT_F = E // NS
RPT = NP // NS

_MESH = plsc.VectorSubcoreMesh(
    core_axis_name="c", subcore_axis_name="s", num_cores=NC, num_subcores=NS)


NBUF = 4
NCH_H = EPT_H // CH
NCH_F = EPT_F // CH


def _cdiv(a, b):
    return (a + b - 1) // b


def _make_stats(ne):
    ept = ne // NW
    nch = ept // CH

    @functools.partial(
        pl.kernel,
        out_type=(
            jax.ShapeDtypeStruct((ne, W), jnp.float32),
            jax.ShapeDtypeStruct((NC, NP, W), jnp.float32),
        ),
        mesh=_MESH,
        scratch_types=(
            pltpu.VMEM((2 * NBUF, CH), jnp.int32),
            pltpu.VMEM((NBUF, CH), jnp.int32),
            pltpu.VMEM((NBUF, CH, W), jnp.float32),
            pltpu.VMEM_SHARED((NP, W), jnp.float32),
            [pltpu.SemaphoreType.DMA] * NBUF,
            [pltpu.SemaphoreType.DMA] * NBUF,
            [pltpu.SemaphoreType.DMA] * (2 * NBUF),
            pltpu.SemaphoreType.DMA,
        ),
    )
    def _gather_stats(ports_hbm, dst_hbm, table_hbm, init_hbm,
                      demb, pe, pring, dring, ring, acc, gsem, dsem, isem,
                      wsem):
        cid = lax.axis_index("c")
        sid = lax.axis_index("s")
        wid = cid * NS + sid
        rows = pl.ds(sid * RPT, RPT)
        pltpu.sync_copy(init_hbm.at[pl.ds(cid * NP + sid * RPT, RPT)],
                        acc.at[rows])
        plsc.subcore_barrier()
        base = wid * ept

        def echunk(k):
            return pl.ds(base + k * CH, CH)

        for j in range(NBUF):
            pltpu.sync_copy(ports_hbm.at[echunk(j)], pring.at[j])
            pltpu.sync_copy(dst_hbm.at[echunk(j)], dring.at[j])
            pltpu.async_copy(table_hbm.at[pring.at[j]], ring.at[j], gsem[j])
            pltpu.async_copy(ports_hbm.at[echunk(j + NBUF)],
                             pring.at[j + NBUF], isem[j + NBUF])

        def body(ko, carry):
            for jj in range(2 * NBUF):
                j = jj % NBUF
                jp = (jj + NBUF) % (2 * NBUF)
                k = ko * 2 * NBUF + jj

                @pl.when(k < nch)
                def _():
                    pltpu.make_async_copy(table_hbm.at[pl.ds(0, CH)],
                                          ring.at[j], gsem[j]).wait()
                    wr = pltpu.async_copy(ring.at[j], demb.at[echunk(k)], wsem)

                    @pl.when(k >= NBUF)
                    def _():
                        pltpu.make_async_copy(dst_hbm.at[pl.ds(0, CH)],
                                              dring.at[j], dsem[j]).wait()

                    pltpu.sync_copy(ring.at[j], acc.at[dring.at[j]], add=True)

                    @pl.when(k + NBUF < nch)
                    def _():
                        pltpu.async_copy(dst_hbm.at[echunk(k + NBUF)],
                                         dring.at[j], dsem[j])

                    wr.wait()

                    @pl.when(k + NBUF < nch)
                    def _():
                        pltpu.make_async_copy(ports_hbm.at[pl.ds(0, CH)],
                                              pring.at[jp], isem[jp]).wait()
                        pltpu.async_copy(table_hbm.at[pring.at[jp]],
                                         ring.at[j], gsem[j])

                    @pl.when(k + 2 * NBUF < nch)
                    def _():
                        pltpu.async_copy(ports_hbm.at[echunk(k + 2 * NBUF)],
                                         pring.at[jj], isem[jj])
            return carry

        lax.fori_loop(0, _cdiv(nch, 2 * NBUF), body, 0)
        plsc.subcore_barrier()
        pltpu.sync_copy(acc.at[rows], pe.at[cid, rows])

    return _gather_stats


EA = 192000
EB = E - EA
_stats_a = _make_stats(EA)
_stats_b = _make_stats(EB)


def _make_scatter(ne):
    ept = ne // NS
    nch = ept // CH

    @functools.partial(
        pl.kernel,
        out_type=jax.ShapeDtypeStruct((NC, NP, W), jnp.float32),
        mesh=_MESH,
        scratch_types=(
            pltpu.VMEM((NBUF, CH), jnp.int32),
            pltpu.VMEM((NBUF, CH, W), jnp.float32),
            pltpu.VMEM_SHARED((NP, W), jnp.float32),
            [pltpu.SemaphoreType.DMA] * NBUF,
            [pltpu.SemaphoreType.DMA] * NBUF,
        ),
    )
    def _scatter_rows(vals_hbm, dst_hbm, init_hbm, out, dring, ring, acc,
                      gsem, dsem):
        cid = lax.axis_index("c")
        sid = lax.axis_index("s")
        rows = pl.ds(sid * RPT, RPT)
        pltpu.sync_copy(init_hbm.at[pl.ds(cid * NP + sid * RPT, RPT)],
                        acc.at[rows])
        plsc.subcore_barrier()
        ebase = sid * ept
        vbase = cid * ne + ebase
        for j in range(NBUF):
            pltpu.sync_copy(dst_hbm.at[pl.ds(ebase + j * CH, CH)], dring.at[j])
            pltpu.async_copy(vals_hbm.at[pl.ds(vbase + j * CH, CH)],
                             ring.at[j], gsem[j])

        def body(ko, carry):
            for j in range(NBUF):
                k = ko * NBUF + j

                @pl.when(k < nch)
                def _():
                    pltpu.make_async_copy(vals_hbm.at[pl.ds(0, CH)],
                                          ring.at[j], gsem[j]).wait()

                    @pl.when(k >= NBUF)
                    def _():
                        pltpu.make_async_copy(dst_hbm.at[pl.ds(0, CH)],
                                              dring.at[j], dsem[j]).wait()

                    pltpu.sync_copy(ring.at[j], acc.at[dring.at[j]], add=True)

                    @pl.when(k + NBUF < nch)
                    def _():
                        pltpu.async_copy(
                            dst_hbm.at[pl.ds(ebase + (k + NBUF) * CH, CH)],
                            dring.at[j], dsem[j])
                        pltpu.async_copy(
                            vals_hbm.at[pl.ds(vbase + (k + NBUF) * CH, CH)],
                            ring.at[j], gsem[j])
            return carry

        lax.fori_loop(0, _cdiv(nch, NBUF), body, 0)
        plsc.subcore_barrier()
        pltpu.sync_copy(acc.at[rows], out.at[cid, rows])

    return _scatter_rows


_scatter_a = _make_scatter(EA)
_scatter_b = _make_scatter(EB)


@functools.partial(
    pl.kernel,
    out_type=jax.ShapeDtypeStruct((NC, NP, W), jnp.float32),
    mesh=_MESH,
    scratch_types=(
        pltpu.VMEM((2 * NBUF, CH), jnp.int32),
        pltpu.VMEM((NBUF, CH), jnp.int32),
        pltpu.VMEM((NBUF, CH, W), jnp.float32),
        pltpu.VMEM_SHARED((NP, W), jnp.float32),
        [pltpu.SemaphoreType.DMA] * NBUF,
        [pltpu.SemaphoreType.DMA] * NBUF,
        [pltpu.SemaphoreType.DMA] * (2 * NBUF),
    ),
)
def _propagate(x_hbm, src2_hbm, dst_hbm, out,
               sring, dring, ring, acc, gsem, dsem, isem):
    cid = lax.axis_index("c")
    sid = lax.axis_index("s")
    rows = pl.ds(sid * RPT, RPT)
    pltpu.sync_copy(x_hbm.at[pl.ds(cid * NP + sid * RPT, RPT)], acc.at[rows])
    plsc.subcore_barrier()
    ebase = sid * EPT_F
    sbase = cid * E + ebase

    for j in range(NBUF):
        pltpu.sync_copy(src2_hbm.at[pl.ds(sbase + j * CH, CH)], sring.at[j])
        pltpu.sync_copy(dst_hbm.at[pl.ds(ebase + j * CH, CH)], dring.at[j])
        pltpu.async_copy(x_hbm.at[sring.at[j]], ring.at[j], gsem[j])
        pltpu.async_copy(src2_hbm.at[pl.ds(sbase + (j + NBUF) * CH, CH)],
                         sring.at[j + NBUF], isem[j + NBUF])

    def body(ko, carry):
        for jj in range(2 * NBUF):
            j = jj % NBUF
            jp = (jj + NBUF) % (2 * NBUF)
            k = ko * 2 * NBUF + jj

            @pl.when(k < NCH_F)
            def _():
                pltpu.make_async_copy(x_hbm.at[pl.ds(0, CH)],
                                      ring.at[j], gsem[j]).wait()

                @pl.when(k >= NBUF)
                def _():
                    pltpu.make_async_copy(dst_hbm.at[pl.ds(0, CH)],
                                          dring.at[j], dsem[j]).wait()

                pltpu.sync_copy(ring.at[j], acc.at[dring.at[j]], add=True)

                @pl.when(k + NBUF < NCH_F)
                def _():
                    pltpu.async_copy(
                        dst_hbm.at[pl.ds(ebase + (k + NBUF) * CH, CH)],
                        dring.at[j], dsem[j])
                    pltpu.make_async_copy(src2_hbm.at[pl.ds(0, CH)],
                                          sring.at[jp], isem[jp]).wait()
                    pltpu.async_copy(x_hbm.at[sring.at[jp]], ring.at[j], gsem[j])

                @pl.when(k + 2 * NBUF < NCH_F)
                def _():
                    pltpu.async_copy(
                        src2_hbm.at[pl.ds(sbase + (k + 2 * NBUF) * CH, CH)],
                        sring.at[jj], isem[jj])
        return carry

    lax.fori_loop(0, _cdiv(NCH_F, 2 * NBUF), body, 0)
    plsc.subcore_barrier()
    pltpu.sync_copy(acc.at[rows], out.at[cid, rows])


def _mlp_math(xa, xe, w1a, w1e, b1, w2, b2, w3, b3):
    h = jnp.dot(xa, w1a, preferred_element_type=jnp.float32)
    h = h + jnp.dot(xe, w1e, preferred_element_type=jnp.float32)
    h = jnp.maximum(h + b1, 0.0)
    h = jnp.maximum(jnp.dot(h, w2, preferred_element_type=jnp.float32) + b2, 0.0)
    return jnp.dot(h, w3, preferred_element_type=jnp.float32) + b3


def _edge_mlp_body(xa, xe, w1a, w1e, b1, w2, b2, w3, b3, out):
    a = xa[...]
    em = _mlp_math(a, xe[...], w1a[...], w1e[...], b1[...],
                   w2[...], b2[...], w3[...], b3[...])
    lo = em[:, :W]
    hi = jnp.concatenate(
        [em[:, W:], a, jnp.zeros((a.shape[0], W - (HID - W) - AD), jnp.float32)],
        axis=1)
    out[...] = jnp.stack([lo, hi])


def _loop_mlp_body(pe, pc, w1a, w1e, b1, w2, b2, w3, b3, out):
    s = pe[0] + pe[1]
    deg = s[:, PD:PD + 1]
    r = 1.0 / jnp.maximum(deg, 1.0)
    xe = s[:, :PD] * r
    xa = pc[1][:, PD:PD + AD] * r
    em = _mlp_math(xa, xe, w1a[...], w1e[...], b1[...],
                   w2[...], b2[...], w3[...], b3[...])
    lo = em[:, :W]
    hi = jnp.concatenate(
        [em[:, W:], jnp.zeros((em.shape[0], 2 * W - HID), jnp.float32)], axis=1)
    out[...] = jnp.stack([lo, hi]) + pc[...]


def _add2_body(a, b, out):
    out[...] = a[...] + b[...]


def _axpy_body(a, b, out):
    out[...] = a[...] + 2.0 * b[...]


def _pool_cls_body(y3, x1, bt, wc1, bc1, wc2, bc2, out, acc):
    i = pl.program_id(0)

    @pl.when(i == 0)
    def _():
        acc[...] = jnp.full((NG, 2 * W), -jnp.inf, jnp.float32)

    x3 = y3[...] + x1[...]
    x = jnp.concatenate([x3[0], x3[1]], axis=1)
    b = bt[...]
    cur = acc[...]
    for g in range(NG):
        xg = jnp.where(b == g, x, -jnp.inf)
        mg = jnp.max(xg, axis=0, keepdims=True)
        seg = (lax.broadcasted_iota(jnp.int32, (NG, 2 * W), 0) == g)
        cur = jnp.where(seg, jnp.maximum(cur, mg), cur)
    acc[...] = cur

    @pl.when(i == pl.num_programs(0) - 1)
    def _():
        gr = acc[...]
        h = jnp.maximum(
            jnp.dot(gr, wc1[...], preferred_element_type=jnp.float32) + bc1[...],
            0.0)
        out[...] = jnp.dot(h, wc2[...], preferred_element_type=jnp.float32) + bc2[...]


_BR_E = 6400
_BR_N = 2048


def _full(shape):
    return pl.BlockSpec(shape, lambda i: (0,) * len(shape))


def _rows(b, w):
    return pl.BlockSpec((b, w), lambda i: (i, 0))


def _rows3(b, w):
    return pl.BlockSpec((NC, b, w), lambda i: (0, i, 0))


def _make_edge_mlp(ne):
    return pl.pallas_call(
        _edge_mlp_body,
        grid=(ne // _BR_E,),
        in_specs=[_rows(_BR_E, AD), _rows(_BR_E, W),
                  _full((AD, HID)), _full((W, HID)), _full((1, HID)),
                  _full((HID, HID)), _full((1, HID)),
                  _full((HID, HID)), _full((1, HID))],
        out_specs=_rows3(_BR_E, W),
        out_shape=jax.ShapeDtypeStruct((NC, ne, W), jnp.float32),
    )


_edge_mlp_a = _make_edge_mlp(EA)
_edge_mlp_b = _make_edge_mlp(EB)

_loop_mlp = pl.pallas_call(
    _loop_mlp_body,
    grid=(NP // _BR_N,),
    in_specs=[_rows3(_BR_N, W), _rows3(_BR_N, W),
              _full((AD, HID)), _full((PD, HID)), _full((1, HID)),
              _full((HID, HID)), _full((1, HID)),
              _full((HID, HID)), _full((1, HID))],
    out_specs=_rows3(_BR_N, W),
    out_shape=jax.ShapeDtypeStruct((NC, NP, W), jnp.float32),
)

_add2 = pl.pallas_call(
    _add2_body,
    grid=(NP // _BR_N,),
    in_specs=[_rows3(_BR_N, W)] * 2,
    out_specs=_rows3(_BR_N, W),
    out_shape=jax.ShapeDtypeStruct((NC, NP, W), jnp.float32),
)

_axpy = pl.pallas_call(
    _axpy_body,
    grid=(NP // _BR_N,),
    in_specs=[_rows3(_BR_N, W)] * 2,
    out_specs=_rows3(_BR_N, W),
    out_shape=jax.ShapeDtypeStruct((NC, NP, W), jnp.float32),
)

_pool_cls = pl.pallas_call(
    _pool_cls_body,
    grid=(NP // _BR_N,),
    in_specs=[_rows3(_BR_N, W)] * 2 + [
        _rows(_BR_N, 1),
        _full((2 * W, HID)), _full((1, HID)),
        _full((HID, NCLS)), _full((1, NCLS))],
    out_specs=_full((NG, NCLS)),
    out_shape=jax.ShapeDtypeStruct((NG, NCLS), jnp.float32),
    scratch_shapes=[pltpu.VMEM((NG, 2 * W), jnp.float32)],
)


def kernel(edge_attr, edge_index, dst_ports, batch, emb_table,
           W1, b1, W2, b2, W3, b3, Wc1, bc1, Wc2, bc2):
    src = edge_index[0]
    dst = edge_index[1]
    zw2 = jnp.zeros((NC * NP, W), jnp.float32)
    table_aug = jnp.concatenate(
        [emb_table, jnp.ones((TP, 1), jnp.float32),
         jnp.zeros((TP, W - PD - 1), jnp.float32)], axis=1)

    demb_a, pe_a = _stats_a(dst_ports[:EA], dst[:EA], table_aug, zw2)
    demb_b, pe = _stats_b(dst_ports[EA:], dst[EA:], table_aug,
                          pe_a.reshape(NC * NP, W))

    w1a = W1[:AD]
    w1e = W1[AD:]
    w1e_pad = jnp.concatenate([w1e, jnp.zeros((W - PD, HID), jnp.float32)])
    b1r, b2r, b3r = b1[None], b2[None], b3[None]

    em2_a = _edge_mlp_a(edge_attr[:EA], demb_a, w1a, w1e_pad, b1r,
                        W2, b2r, W3, b3r)
    em2_b = _edge_mlp_b(edge_attr[EA:], demb_b, w1a, w1e_pad, b1r,
                        W2, b2r, W3, b3r)
    p1 = _scatter_a(em2_a.reshape(NC * EA, W), dst[:EA], zw2)
    pc = _scatter_b(em2_b.reshape(NC * EB, W), dst[EA:],
                    p1.reshape(NC * NP, W))
    em_n2 = _loop_mlp(pe, pc, w1a, w1e, b1r, W2, b2r, W3, b3r)

    x1 = em_n2
    src2 = jnp.concatenate([src, src + NP])
    y2 = _propagate(x1.reshape(NC * NP, W), src2, dst)
    x2 = _add2(y2, x1)
    y3 = _propagate(x2.reshape(NC * NP, W), src2, dst)

    batch_p = jnp.concatenate([batch, jnp.full((NP - N,), NG, jnp.int32)])
    wc1_pad = jnp.concatenate([Wc1, jnp.zeros((2 * W - HID, HID), jnp.float32)])
    out = _pool_cls(y3, x1, batch_p[:, None],
                    wc1_pad, bc1[None], Wc2, bc2[None])
    return out
